# Initial kernel scaffold; baseline (speedup 1.0000x reference)
import jax
import jax.numpy as jnp
from jax.experimental import pallas as pl


def kernel(observations, actions, embeddings, positional_emb):
    raise NotImplementedError("write your pallas kernel here")


# speedup vs baseline: 1.9645x; 1.9645x over previous
import functools

import jax
import jax.numpy as jnp
from jax import lax
from jax.experimental import pallas as pl
from jax.experimental.pallas import tpu as pltpu
from jax.experimental.pallas import tpu_sc as plsc

_MAX_OBS_TOKENS = 512
_LANES = 16

_NUM_CORES = 2
_NUM_SUBCORES = 16
_NUM_WORKERS = _NUM_CORES * _NUM_SUBCORES


def _lookup_add(idx, table, pos_pat, n_rows, d, g_rows, chunk, gather_slice):
    per_w = n_rows // _NUM_WORKERS
    nchunks = per_w // chunk
    n_gathers = chunk // gather_slice
    groups = chunk // g_rows
    vregs_per_row = d // _LANES
    assert nchunks % 2 == 0

    mesh = plsc.VectorSubcoreMesh(
        core_axis_name="c", subcore_axis_name="s")

    @functools.partial(
        pl.kernel,
        out_type=jax.ShapeDtypeStruct((n_rows, d), jnp.float32),
        mesh=mesh,
        scratch_types=[
            pltpu.VMEM((per_w,), jnp.int32),
            pltpu.VMEM((chunk, d), jnp.float32),
            pltpu.VMEM((chunk, d), jnp.float32),
            pltpu.VMEM((g_rows, d), jnp.float32),
            pltpu.SemaphoreType.DMA,
            pltpu.SemaphoreType.DMA,
            pltpu.SemaphoreType.DMA,
            pltpu.SemaphoreType.DMA,
        ],
        compiler_params=pltpu.CompilerParams(use_tc_tiling_on_sc=False),
    )
    def k(idx_hbm, table_hbm, pos_hbm, out_hbm,
          idx_v, rows0, rows1, pos_v, gsem0, gsem1, wsem0, wsem1):
        rows = (rows0, rows1)
        gsem = (gsem0, gsem1)
        wsem = (wsem0, wsem1)
        wid = lax.axis_index("s") * _NUM_CORES + lax.axis_index("c")
        base_w = pl.multiple_of(wid * per_w, 8)
        pltpu.sync_copy(idx_hbm.at[pl.ds(base_w, per_w)], idx_v)
        pltpu.sync_copy(pos_hbm, pos_v)

        def fire_gathers(c, b):
            for j in range(n_gathers):
                pltpu.async_copy(
                    table_hbm.at[idx_v.at[pl.ds(
                        c * chunk + j * gather_slice, gather_slice)]],
                    rows[b].at[pl.ds(j * gather_slice, gather_slice)],
                    gsem[b])

        def drain_gathers(b):
            pltpu.make_async_copy(
                out_hbm.at[pl.ds(0, chunk)], rows[b], gsem[b]).wait()

        def fire_write(c, b):
            dst = out_hbm.at[pl.ds(
                pl.multiple_of(base_w + c * chunk, 8), chunk)]
            pltpu.async_copy(rows[b], dst, wsem[b])

        def drain_write(b):
            pltpu.make_async_copy(
                out_hbm.at[pl.ds(0, chunk)], rows[b], wsem[b]).wait()

        def add_pos(b):
            def group_body(g, _):
                base = g * g_rows
                for t in range(g_rows):
                    for jv in range(vregs_per_row):
                        plsc.addupdate(
                            rows[b].at[base + t, pl.ds(jv * _LANES, _LANES)],
                            pos_v[t, pl.ds(jv * _LANES, _LANES)])
                return 0
            lax.fori_loop(0, groups, group_body, 0)

        def pair_body(jj, _):
            i2 = jj * 2
            for b in (0, 1):
                c = i2 + b

                @pl.when(c >= 2)
                def _():
                    drain_write(b)

                fire_gathers(c, b)
                b1 = 1 - b

                @pl.when(c >= 1)
                def _():
                    drain_gathers(b1)
                    add_pos(b1)
                    fire_write(c - 1, b1)
            return 0

        lax.fori_loop(0, nchunks // 2, pair_body, 0)
        drain_gathers(1)
        add_pos(1)
        fire_write(nchunks - 1, 1)
        drain_write(0)
        drain_write(1)

    return k(idx, table, pos_pat)


def kernel(observations, actions, embeddings, positional_emb):
    B, L, T = observations.shape
    A = actions.shape[2]
    d = embeddings.shape[1]
    G = T + 1 + A
    separator = embeddings.shape[0] - 1
    N = B * L * G

    sep_col = jnp.full((B, L, 1), separator, dtype=jnp.int32)
    idx = jnp.concatenate(
        [observations.astype(jnp.int32), sep_col, actions.astype(jnp.int32)],
        axis=2).reshape(-1)

    pos_pat = jnp.concatenate([
        positional_emb[:T],
        jnp.zeros((1, d), jnp.float32),
        jnp.broadcast_to(positional_emb[_MAX_OBS_TOKENS][None, :], (A, d)),
    ], axis=0)

    out = _lookup_add(idx, embeddings, pos_pat, N, d, G,
                      chunk=16 * G, gather_slice=80)
    return out.reshape(B, L * G, d)
